# baseline (speedup 1.0000x reference)
import functools

import jax
import jax.numpy as jnp
from jax import lax
from jax.experimental import pallas as pl
from jax.experimental.pallas import tpu as pltpu
from jax.experimental.pallas import tpu_sc as plsc

B = 16384
D = 1024
NC = 2
NS = 16
NW = NC * NS
BPW = B // NW
C = 16
NCHUNK = BPW // C
NBUF = 4
PRIME = 2
NGROUP = NCHUNK // NBUF
WPR = 4096 // BPW
LANES = 16
SCALE = 32.0

_mesh = plsc.VectorSubcoreMesh(core_axis_name="c", subcore_axis_name="s")


@functools.partial(
    pl.kernel,
    mesh=_mesh,
    out_type=jax.ShapeDtypeStruct((4, 4096, D), jnp.float32),
    scratch_types=[
        pltpu.VMEM((BPW,), jnp.int32),
    ]
    + [pltpu.VMEM((C, D), jnp.float32) for _ in range(NBUF)]
    + [pltpu.SemaphoreType.DMA for _ in range(2 * NBUF)],
)
def _embed(idx_hbm, table_hbm, out_hbm, idx_v, *rest):
    bufs = rest[:NBUF]
    gsems = rest[NBUF : 2 * NBUF]
    ssems = rest[2 * NBUF :]
    wid = lax.axis_index("s") * NC + lax.axis_index("c")
    row = wid // WPR
    off = (wid % WPR) * BPW
    pltpu.sync_copy(idx_hbm.at[row, pl.ds(off, BPW)], idx_v)

    def gather(c, b):
        return pltpu.async_copy(
            table_hbm.at[idx_v.at[pl.ds(c * C, C)]], bufs[b], gsems[b]
        )

    def scatter(c, b):
        return pltpu.async_copy(
            bufs[b], out_hbm.at[row, pl.ds(off + c * C, C)], ssems[b]
        )

    def wait_scatter(c, b):
        pltpu.make_async_copy(
            bufs[b], out_hbm.at[row, pl.ds(off + c * C, C)], ssems[b]
        ).wait()

    def wait_gather(c, b):
        pltpu.make_async_copy(
            table_hbm.at[idx_v.at[pl.ds(c * C, C)]], bufs[b], gsems[b]
        ).wait()

    def scale(buf):
        def scale_rows(j, carry):
            for r in range(2):
                for k in range(D // LANES):
                    sl = pl.ds(k * LANES, LANES)
                    buf[2 * j + r, sl] = buf[2 * j + r, sl] * SCALE
            return carry

        lax.fori_loop(0, C // 2, scale_rows, 0)

    for c in range(PRIME):
        gather(c, c)

    def group(t, carry):
        for b in range(NBUF):
            c = t * NBUF + b
            b2 = (b + PRIME) % NBUF
            g = c + PRIME

            @pl.when(g < NCHUNK)
            def _():
                @pl.when(g >= NBUF)
                def _():
                    wait_scatter(g - NBUF, b2)

                gather(g, b2)

            wait_gather(c, b)
            scale(bufs[b])
            scatter(c, b)
        return carry

    lax.fori_loop(0, NGROUP, group, 0)

    for b in range(NBUF):
        wait_scatter((NGROUP - 1) * NBUF + b, b)


def kernel(token_ids, table):
    return _embed(token_ids, table)


# speedup vs baseline: 1.1631x; 1.1631x over previous
import functools

import jax
import jax.numpy as jnp
from jax import lax
from jax.experimental import pallas as pl
from jax.experimental.pallas import tpu as pltpu
from jax.experimental.pallas import tpu_sc as plsc

B = 16384
D = 1024
NC = 2
NS = 16
NW = NC * NS
BPW = B // NW
C = 16
NCHUNK = BPW // C
NBUF = 4
PRIME = 2
NGROUP = NCHUNK // NBUF
WPR = 4096 // BPW
LANES = 16
SCALE = 32.0

_mesh = plsc.VectorSubcoreMesh(core_axis_name="c", subcore_axis_name="s")


@functools.partial(
    pl.kernel,
    mesh=_mesh,
    out_type=jax.ShapeDtypeStruct((4, 4096, D), jnp.float32),
    scratch_types=[
        pltpu.VMEM((BPW,), jnp.int32),
    ]
    + [pltpu.VMEM((C, D), jnp.float32) for _ in range(NBUF)]
    + [pltpu.SemaphoreType.DMA for _ in range(2 * NBUF)],
)
def _embed(idx_hbm, table_hbm, out_hbm, idx_v, *rest):
    bufs = rest[:NBUF]
    gsems = rest[NBUF : 2 * NBUF]
    ssems = rest[2 * NBUF :]
    wid = lax.axis_index("s") * NC + lax.axis_index("c")
    row = wid // WPR
    off = (wid % WPR) * BPW
    pltpu.sync_copy(idx_hbm.at[row, pl.ds(off, BPW)], idx_v)

    def gather(c, b):
        return pltpu.async_copy(
            table_hbm.at[idx_v.at[pl.ds(c * C, C)]], bufs[b], gsems[b]
        )

    def scatter(c, b):
        return pltpu.async_copy(
            bufs[b], out_hbm.at[row, pl.ds(off + c * C, C)], ssems[b]
        )

    def wait_scatter(c, b):
        pltpu.make_async_copy(
            bufs[b], out_hbm.at[row, pl.ds(off + c * C, C)], ssems[b]
        ).wait()

    def wait_gather(c, b):
        pltpu.make_async_copy(
            table_hbm.at[idx_v.at[pl.ds(c * C, C)]], bufs[b], gsems[b]
        ).wait()

    def scale(buf):
        def scale_halfrow(m, carry):
            j = m >> 1
            colbase = (m & 1) * (D // 2)
            for k in range(D // (2 * LANES)):
                sl = pl.ds(colbase + k * LANES, LANES)
                buf[j, sl] = buf[j, sl] * SCALE
            return carry

        lax.fori_loop(0, 2 * C, scale_halfrow, 0)

    for c in range(PRIME):
        gather(c, c)

    def group(t, carry):
        for b in range(NBUF):
            c = t * NBUF + b
            b2 = (b + PRIME) % NBUF
            g = c + PRIME

            @pl.when(g < NCHUNK)
            def _():
                @pl.when(g >= NBUF)
                def _():
                    wait_scatter(g - NBUF, b2)

                gather(g, b2)

            wait_gather(c, b)
            scale(bufs[b])
            scatter(c, b)
        return carry

    lax.fori_loop(0, NGROUP, group, 0)

    for b in range(NBUF):
        wait_scatter((NGROUP - 1) * NBUF + b, b)


def kernel(token_ids, table):
    return _embed(token_ids, table)
